# baseline (device time: 49991 ns/iter reference)
import jax
import jax.numpy as jnp
from jax import lax
from jax.experimental import pallas as pl
from jax.experimental.pallas import tpu as pltpu

N_DEV = 8
SUB = 4


def kernel(x, w_mat):
    m, k_per = x.shape
    n = w_mat.shape[1]
    m_out = m // N_DEV
    nh = n // 2
    cs = nh // SUB

    xb = x.astype(jnp.bfloat16)
    wb = w_mat.astype(jnp.bfloat16)

    gelu_c = 0.7978845608028654

    def gelu(a):
        return 0.5 * a * (1.0 + jnp.tanh(gelu_c * (a + 0.044715 * a * a * a)))

    def body(x_ref, w_ref, out_ref,
             stage_r, stage_l, recv_r, recv_l,
             send_sems_r, recv_sems_r, send_sems_l, recv_sems_l):
        my = lax.axis_index("i")
        left = lax.rem(my + (N_DEV - 1), N_DEV)
        right = lax.rem(my + 1, N_DEV)

        barrier = pltpu.get_barrier_semaphore()
        for nbr in (left, right):
            pl.semaphore_signal(
                barrier, inc=1, device_id=(nbr,),
                device_id_type=pl.DeviceIdType.MESH,
            )

        def partial(c, col0, col1, dtype=jnp.bfloat16):
            return jnp.dot(
                x_ref[pl.ds(c * m_out, m_out), :],
                w_ref[:, col0:col1],
                preferred_element_type=jnp.float32,
            ).astype(dtype)

        def make(s, q, stage, recv, ssems, rsems, tgt):
            return pltpu.make_async_remote_copy(
                src_ref=stage.at[s % 2, q],
                dst_ref=recv.at[s, q],
                send_sem=ssems.at[s, q],
                recv_sem=rsems.at[s, q],
                device_id=(tgt,),
                device_id_type=pl.DeviceIdType.MESH,
            )

        def make_r(s, q):
            return make(s, q, stage_r, recv_r, send_sems_r, recv_sems_r, right)

        def make_l(s, q):
            return make(s, q, stage_l, recv_l, send_sems_l, recv_sems_l, left)

        c_r0 = lax.rem(my + (N_DEV - 1), N_DEV)
        c_l0 = lax.rem(my + 1, N_DEV)
        for q in range(SUB):
            stage_r[0, q] = partial(c_r0, q * cs, (q + 1) * cs)
            stage_l[0, q] = partial(c_l0, nh + q * cs, nh + (q + 1) * cs)
        pl.semaphore_wait(barrier, 2)
        for q in range(SUB):
            make_r(0, q).start()
            make_l(0, q).start()

        for s in range(N_DEV - 1):
            last = s == N_DEV - 2
            c_r = lax.rem(my + (2 * N_DEV - 2 - s), N_DEV)
            c_l = lax.rem(my + 2 + s, N_DEV)
            acc_t = jnp.float32 if last else jnp.bfloat16
            p_r = partial(c_r, 0, nh, acc_t)
            p_l = partial(c_l, nh, n, acc_t)
            for q in range(SUB):
                qs = slice(q * cs, (q + 1) * cs)
                make_r(s, q).wait_recv()
                acc_r = recv_r[s, q].astype(acc_t) + p_r[:, qs]
                if not last:
                    if s >= 1:
                        make_r(s - 1, q).wait_send()
                    stage_r[(s + 1) % 2, q] = acc_r
                    make_r(s + 1, q).start()
                else:
                    out_ref[:, qs] = gelu(acc_r).astype(jnp.bfloat16)
                make_l(s, q).wait_recv()
                acc_l = recv_l[s, q].astype(acc_t) + p_l[:, qs]
                if not last:
                    if s >= 1:
                        make_l(s - 1, q).wait_send()
                    stage_l[(s + 1) % 2, q] = acc_l
                    make_l(s + 1, q).start()
                else:
                    out_ref[:, nh + q * cs:nh + (q + 1) * cs] = gelu(
                        acc_l).astype(jnp.bfloat16)

        for s in (N_DEV - 3, N_DEV - 2):
            for q in range(SUB):
                make_r(s, q).wait_send()
                make_l(s, q).wait_send()

    return pl.pallas_call(
        body,
        out_shape=jax.ShapeDtypeStruct((m_out, n), jnp.bfloat16),
        in_specs=[
            pl.BlockSpec(memory_space=pltpu.VMEM),
            pl.BlockSpec(memory_space=pltpu.VMEM),
        ],
        out_specs=pl.BlockSpec(memory_space=pltpu.VMEM),
        scratch_shapes=[
            pltpu.VMEM((2, SUB, m_out, cs), jnp.bfloat16),
            pltpu.VMEM((2, SUB, m_out, cs), jnp.bfloat16),
            pltpu.VMEM((N_DEV - 1, SUB, m_out, cs), jnp.bfloat16),
            pltpu.VMEM((N_DEV - 1, SUB, m_out, cs), jnp.bfloat16),
            pltpu.SemaphoreType.DMA((N_DEV - 1, SUB)),
            pltpu.SemaphoreType.DMA((N_DEV - 1, SUB)),
            pltpu.SemaphoreType.DMA((N_DEV - 1, SUB)),
            pltpu.SemaphoreType.DMA((N_DEV - 1, SUB)),
        ],
        compiler_params=pltpu.CompilerParams(collective_id=0),
    )(xb, wb)


# device time: 49676 ns/iter; 1.0063x vs baseline; 1.0063x over previous
import jax
import jax.numpy as jnp
from jax import lax
from jax.experimental import pallas as pl
from jax.experimental.pallas import tpu as pltpu

N_DEV = 8
SUB = 2


def kernel(x, w_mat):
    m, k_per = x.shape
    n = w_mat.shape[1]
    m_out = m // N_DEV
    nh = n // 2
    cs = nh // SUB

    xb = x.astype(jnp.bfloat16)
    wb = w_mat.astype(jnp.bfloat16)

    gelu_c = 0.7978845608028654

    def gelu(a):
        return 0.5 * a * (1.0 + jnp.tanh(gelu_c * (a + 0.044715 * a * a * a)))

    def body(x_ref, w_ref, out_ref,
             stage_r, stage_l, recv_r, recv_l,
             send_sems_r, recv_sems_r, send_sems_l, recv_sems_l):
        my = lax.axis_index("i")
        left = lax.rem(my + (N_DEV - 1), N_DEV)
        right = lax.rem(my + 1, N_DEV)

        barrier = pltpu.get_barrier_semaphore()
        for nbr in (left, right):
            pl.semaphore_signal(
                barrier, inc=1, device_id=(nbr,),
                device_id_type=pl.DeviceIdType.MESH,
            )

        def partial(c, col0, col1, dtype=jnp.bfloat16):
            return jnp.dot(
                x_ref[pl.ds(c * m_out, m_out), :],
                w_ref[:, col0:col1],
                preferred_element_type=jnp.float32,
            ).astype(dtype)

        def make(s, q, stage, recv, ssems, rsems, tgt):
            return pltpu.make_async_remote_copy(
                src_ref=stage.at[s % 2, q],
                dst_ref=recv.at[s, q],
                send_sem=ssems.at[s, q],
                recv_sem=rsems.at[s, q],
                device_id=(tgt,),
                device_id_type=pl.DeviceIdType.MESH,
            )

        def make_r(s, q):
            return make(s, q, stage_r, recv_r, send_sems_r, recv_sems_r, right)

        def make_l(s, q):
            return make(s, q, stage_l, recv_l, send_sems_l, recv_sems_l, left)

        c_r0 = lax.rem(my + (N_DEV - 1), N_DEV)
        c_l0 = lax.rem(my + 1, N_DEV)
        for q in range(SUB):
            stage_r[0, q] = partial(c_r0, q * cs, (q + 1) * cs)
            stage_l[0, q] = partial(c_l0, nh + q * cs, nh + (q + 1) * cs)
        pl.semaphore_wait(barrier, 2)
        for q in range(SUB):
            make_r(0, q).start()
            make_l(0, q).start()

        for s in range(N_DEV - 1):
            last = s == N_DEV - 2
            c_r = lax.rem(my + (2 * N_DEV - 2 - s), N_DEV)
            c_l = lax.rem(my + 2 + s, N_DEV)
            acc_t = jnp.float32 if last else jnp.bfloat16
            p_r = partial(c_r, 0, nh, acc_t)
            p_l = partial(c_l, nh, n, acc_t)
            for q in range(SUB):
                qs = slice(q * cs, (q + 1) * cs)
                make_r(s, q).wait_recv()
                acc_r = recv_r[s, q].astype(acc_t) + p_r[:, qs]
                if not last:
                    if s >= 1:
                        make_r(s - 1, q).wait_send()
                    stage_r[(s + 1) % 2, q] = acc_r
                    make_r(s + 1, q).start()
                else:
                    out_ref[:, qs] = gelu(acc_r).astype(jnp.bfloat16)
                make_l(s, q).wait_recv()
                acc_l = recv_l[s, q].astype(acc_t) + p_l[:, qs]
                if not last:
                    if s >= 1:
                        make_l(s - 1, q).wait_send()
                    stage_l[(s + 1) % 2, q] = acc_l
                    make_l(s + 1, q).start()
                else:
                    out_ref[:, nh + q * cs:nh + (q + 1) * cs] = gelu(
                        acc_l).astype(jnp.bfloat16)

        for s in (N_DEV - 3, N_DEV - 2):
            for q in range(SUB):
                make_r(s, q).wait_send()
                make_l(s, q).wait_send()

    return pl.pallas_call(
        body,
        out_shape=jax.ShapeDtypeStruct((m_out, n), jnp.bfloat16),
        in_specs=[
            pl.BlockSpec(memory_space=pltpu.VMEM),
            pl.BlockSpec(memory_space=pltpu.VMEM),
        ],
        out_specs=pl.BlockSpec(memory_space=pltpu.VMEM),
        scratch_shapes=[
            pltpu.VMEM((2, SUB, m_out, cs), jnp.bfloat16),
            pltpu.VMEM((2, SUB, m_out, cs), jnp.bfloat16),
            pltpu.VMEM((N_DEV - 1, SUB, m_out, cs), jnp.bfloat16),
            pltpu.VMEM((N_DEV - 1, SUB, m_out, cs), jnp.bfloat16),
            pltpu.SemaphoreType.DMA((N_DEV - 1, SUB)),
            pltpu.SemaphoreType.DMA((N_DEV - 1, SUB)),
            pltpu.SemaphoreType.DMA((N_DEV - 1, SUB)),
            pltpu.SemaphoreType.DMA((N_DEV - 1, SUB)),
        ],
        compiler_params=pltpu.CompilerParams(collective_id=0),
    )(xb, wb)
